# baseline (speedup 1.0000x reference)
import jax
import jax.numpy as jnp
from jax.experimental import pallas as pl
from jax.experimental.pallas import tpu as pltpu

N_USERS = 4096
X_DIM = 64
K_SUBS = 20000
R_DIM = 3

BM = 32
NM = N_USERS // BM


def _body(x_ref, s_ref, rt_ref, o_ref):
    o_ref[:, :X_DIM] = x_ref[...]
    s = s_ref[...]
    for j in range(R_DIM):
        rj = rt_ref[j : j + 1, :]
        o_ref[:, X_DIM + j : X_DIM + j + 1] = jnp.sum(
            s * rj, axis=1, keepdims=True
        )


def kernel(x, S, R):
    return pl.pallas_call(
        _body,
        grid=(NM,),
        in_specs=[
            pl.BlockSpec((BM, X_DIM), lambda m: (m, 0)),
            pl.BlockSpec((BM, K_SUBS), lambda m: (m, 0)),
            pl.BlockSpec((R_DIM, K_SUBS), lambda m: (0, 0)),
        ],
        out_specs=pl.BlockSpec((BM, X_DIM + R_DIM), lambda m: (m, 0)),
        out_shape=jax.ShapeDtypeStruct((N_USERS, X_DIM + R_DIM), jnp.float32),
        compiler_params=pltpu.CompilerParams(
            dimension_semantics=("arbitrary",),
        ),
    )(x, S, R.T)


# speedup vs baseline: 1.1414x; 1.1414x over previous
import jax
import jax.numpy as jnp
from jax.experimental import pallas as pl
from jax.experimental.pallas import tpu as pltpu

N_USERS = 4096
X_DIM = 64
K_SUBS = 20000
R_DIM = 3

NSPLIT = 8
BM = 32
NM = N_USERS // (BM * NSPLIT)
MQ = N_USERS // NSPLIT


def _body(*refs):
    s_refs = refs[:NSPLIT]
    rt_ref = refs[NSPLIT]
    o_refs = refs[NSPLIT + 1 :]
    for q in range(NSPLIT):
        s = s_refs[q][...]
        for j in range(R_DIM):
            o_refs[q][:, j : j + 1] = jnp.sum(
                s * rt_ref[j : j + 1, :], axis=1, keepdims=True
            )


def kernel(x, S, R):
    rt = R.T
    s_specs = [
        pl.BlockSpec((BM, K_SUBS), lambda m, q=q: (q * NM + m, 0))
        for q in range(NSPLIT)
    ]
    aggs = pl.pallas_call(
        _body,
        grid=(NM,),
        in_specs=[
            *s_specs,
            pl.BlockSpec((R_DIM, K_SUBS), lambda m: (0, 0)),
        ],
        out_specs=[
            pl.BlockSpec((BM, R_DIM), lambda m: (m, 0))
            for _ in range(NSPLIT)
        ],
        out_shape=[
            jax.ShapeDtypeStruct((MQ, R_DIM), jnp.float32)
            for _ in range(NSPLIT)
        ],
        compiler_params=pltpu.CompilerParams(
            dimension_semantics=("arbitrary",),
        ),
    )(*([S] * NSPLIT), rt)
    return jnp.concatenate([x, jnp.concatenate(aggs, axis=0)], axis=1)


# speedup vs baseline: 3.3115x; 2.9012x over previous
import jax
import jax.numpy as jnp
from jax.experimental import pallas as pl
from jax.experimental.pallas import tpu as pltpu

N_USERS = 4096
X_DIM = 64
K_SUBS = 20000
R_DIM = 3

BK = 400
NK = K_SUBS // BK


def _body(st_ref, r_ref, o_ref, acc_ref):
    k = pl.program_id(0)

    @pl.when(k == 0)
    def _init():
        acc_ref[...] = jnp.zeros_like(acc_ref)

    st = st_ref[...]
    for j in range(R_DIM):
        acc_ref[j : j + 1, :] += jnp.sum(
            st * r_ref[:, j : j + 1], axis=0, keepdims=True
        )

    @pl.when(k == NK - 1)
    def _fin():
        o_ref[...] = acc_ref[...]


def kernel(x, S, R):
    agg_t = pl.pallas_call(
        _body,
        grid=(NK,),
        in_specs=[
            pl.BlockSpec((BK, N_USERS), lambda k: (k, 0)),
            pl.BlockSpec((BK, R_DIM), lambda k: (k, 0)),
        ],
        out_specs=pl.BlockSpec((R_DIM, N_USERS), lambda k: (0, 0)),
        out_shape=jax.ShapeDtypeStruct((R_DIM, N_USERS), jnp.float32),
        scratch_shapes=[pltpu.VMEM((R_DIM, N_USERS), jnp.float32)],
        compiler_params=pltpu.CompilerParams(
            dimension_semantics=("arbitrary",),
        ),
    )(S.T, R)
    return jnp.concatenate([x, agg_t.T], axis=1)


# speedup vs baseline: 3.5850x; 1.0826x over previous
import jax
import jax.numpy as jnp
from jax.experimental import pallas as pl
from jax.experimental.pallas import tpu as pltpu

N_USERS = 4096
X_DIM = 64
K_SUBS = 20000
R_DIM = 3

BK = 800
NK = K_SUBS // BK


def _body(st_ref, r_ref, o_ref, acc_ref):
    k = pl.program_id(0)

    @pl.when(k == 0)
    def _init():
        acc_ref[...] = jnp.zeros_like(acc_ref)

    st = st_ref[...]
    for j in range(R_DIM):
        acc_ref[j : j + 1, :] += jnp.sum(
            st * r_ref[:, j : j + 1], axis=0, keepdims=True
        )

    @pl.when(k == NK - 1)
    def _fin():
        o_ref[...] = acc_ref[...]


def kernel(x, S, R):
    agg_t = pl.pallas_call(
        _body,
        grid=(NK,),
        in_specs=[
            pl.BlockSpec((BK, N_USERS), lambda k: (k, 0)),
            pl.BlockSpec((BK, R_DIM), lambda k: (k, 0)),
        ],
        out_specs=pl.BlockSpec((R_DIM, N_USERS), lambda k: (0, 0)),
        out_shape=jax.ShapeDtypeStruct((R_DIM, N_USERS), jnp.float32),
        scratch_shapes=[pltpu.VMEM((R_DIM, N_USERS), jnp.float32)],
        compiler_params=pltpu.CompilerParams(
            dimension_semantics=("arbitrary",),
        ),
    )(S.T, R)
    return jnp.concatenate([x, agg_t.T], axis=1)
